# baseline (device time: 71301 ns/iter reference)
import jax
import jax.numpy as jnp
from jax import lax
from jax.experimental import pallas as pl
from jax.experimental.pallas import tpu as pltpu

N_DEV = 16
F8 = jnp.float8_e4m3fn
G = 4
NB = 8
DEV_PER_G = N_DEV // G


def kernel(x, w_mat, scale_x, scale_w):
    m_total, k_shard = x.shape
    k_total, n_total = w_mat.shape
    m_blk = m_total // N_DEV
    k_grp = k_total // G
    n_blk = n_total // NB

    def body(gorder_ref, x_ref, w_ref, sx_ref, sw_ref, out_ref,
             x8_ref, recv_ref, xg_ref, send_sems, recv_sems):
        g = pl.program_id(0)
        n = pl.program_id(1)
        me = lax.axis_index("i")
        mz = me // DEV_PER_G
        ga = gorder_ref[g]

        @pl.when(jnp.logical_and(g == 0, n == 0))
        def _setup():
            x8_ref[...] = x_ref[...].astype(F8)

            bsem = pltpu.get_barrier_semaphore()
            for d in range(N_DEV):
                pl.semaphore_signal(
                    bsem, inc=1,
                    device_id=(d,), device_id_type=pl.DeviceIdType.MESH,
                )
            pl.semaphore_wait(bsem, N_DEV)

            for t in range(G):
                rz = (mz - t) % G
                for q in range(DEV_PER_G):
                    d = rz * DEV_PER_G + q

                    @pl.when(d != me)
                    def _send():
                        rdma = pltpu.make_async_remote_copy(
                            src_ref=x8_ref.at[pl.ds(d * m_blk, m_blk), :],
                            dst_ref=recv_ref.at[me],
                            send_sem=send_sems.at[d],
                            recv_sem=recv_sems.at[me],
                            device_id=(d,),
                            device_id_type=pl.DeviceIdType.MESH,
                        )
                        rdma.start()

            xg_ref[:, pl.ds(me * m_blk, m_blk)] = (
                x8_ref[pl.ds(me * m_blk, m_blk), :]
            )

        @pl.when(n == 0)
        def _collect_group():
            for q in range(DEV_PER_G):
                j = ga * DEV_PER_G + q

                @pl.when(j != me)
                def _recv():
                    recv = pltpu.make_async_remote_copy(
                        src_ref=x8_ref.at[pl.ds(0, m_blk), :],
                        dst_ref=recv_ref.at[j],
                        send_sem=send_sems.at[j],
                        recv_sem=recv_sems.at[j],
                        device_id=(me,),
                        device_id_type=pl.DeviceIdType.MESH,
                    )
                    recv.wait_recv()
                    xg_ref[:, pl.ds(j * m_blk, m_blk)] = recv_ref[j]

        a = xg_ref[:, pl.ds(ga * k_grp, k_grp)]
        w8 = w_ref[...].astype(F8)
        part = jnp.dot(a, w8, preferred_element_type=jnp.float32)
        nsl = pl.ds(n * n_blk, n_blk)

        @pl.when(g == 0)
        def _init():
            out_ref[:, nsl] = part

        @pl.when(jnp.logical_and(g != 0, g != G - 1))
        def _accum():
            out_ref[:, nsl] += part

        @pl.when(g == G - 1)
        def _fin():
            s = sx_ref[0] * sw_ref[0]
            out_ref[:, nsl] = jnp.maximum((out_ref[:, nsl] + part) * s, 0.0)

        @pl.when(jnp.logical_and(g == G - 1, n == NB - 1))
        def _drain():
            for d in range(N_DEV):
                @pl.when(d != me)
                def _d():
                    sd = pltpu.make_async_remote_copy(
                        src_ref=x8_ref.at[pl.ds(d * m_blk, m_blk), :],
                        dst_ref=recv_ref.at[me],
                        send_sem=send_sems.at[d],
                        recv_sem=recv_sems.at[me],
                        device_id=(d,),
                        device_id_type=pl.DeviceIdType.MESH,
                    )
                    sd.wait_send()

    grid_spec = pltpu.PrefetchScalarGridSpec(
        num_scalar_prefetch=1,
        grid=(G, NB),
        in_specs=[
            pl.BlockSpec((m_total, k_shard), lambda g, n, go: (0, 0)),
            pl.BlockSpec((k_grp, n_blk), lambda g, n, go: (go[g], n)),
            pl.BlockSpec(memory_space=pltpu.SMEM),
            pl.BlockSpec(memory_space=pltpu.SMEM),
        ],
        out_specs=pl.BlockSpec((m_blk, n_total), lambda g, n, go: (0, 0)),
        scratch_shapes=[
            pltpu.VMEM((m_total, k_shard), F8),
            pltpu.VMEM((N_DEV, m_blk, k_shard), F8),
            pltpu.VMEM((m_blk, k_total), F8),
            pltpu.SemaphoreType.DMA((N_DEV,)),
            pltpu.SemaphoreType.DMA((N_DEV,)),
        ],
    )

    me = lax.axis_index("i")
    gorder = (me // DEV_PER_G + jnp.arange(G, dtype=jnp.int32)) % G

    return pl.pallas_call(
        body,
        grid_spec=grid_spec,
        out_shape=jax.ShapeDtypeStruct((m_blk, n_total), jnp.float32),
        compiler_params=pltpu.CompilerParams(
            dimension_semantics=("arbitrary", "arbitrary"),
            collective_id=0,
            has_side_effects=True,
            vmem_limit_bytes=64 * 1024 * 1024,
        ),
    )(gorder, x, w_mat, scale_x, scale_w)


# device time: 59040 ns/iter; 1.2077x vs baseline; 1.2077x over previous
import jax
import jax.numpy as jnp
from jax import lax
from jax.experimental import pallas as pl
from jax.experimental.pallas import tpu as pltpu

N_DEV = 16
F8 = jnp.float8_e4m3fn
R = 4
S = 4
H = 2


def kernel(x, w_mat, scale_x, scale_w):
    m_total, k_shard = x.shape
    k_total, n_total = w_mat.shape
    m_blk = m_total // N_DEV
    rows = m_blk // S
    n_h = n_total // H

    def body(x_ref, w_ref, sx_ref, sw_ref, out_ref,
             wbuf_ref, x8_ref, recv_ref, wsems, send_sems, recv_sems):
        c = pl.program_id(0)
        me = lax.axis_index("i")

        def fill_slot(slot, chunk):
            for i in range(S):
                pltpu.make_async_copy(
                    w_ref.at[pl.ds(chunk * m_blk + i * rows, rows), :],
                    wbuf_ref.at[slot, pl.ds(i * rows, rows), :],
                    wsems.at[slot, i],
                ).start()

        def wait_slot(slot, chunk):
            for i in range(S):
                pltpu.make_async_copy(
                    w_ref.at[pl.ds(chunk * m_blk + i * rows, rows), :],
                    wbuf_ref.at[slot, pl.ds(i * rows, rows), :],
                    wsems.at[slot, i],
                ).wait()

        @pl.when(c == 0)
        def _setup():
            for r in range(R):
                fill_slot(r, (me + r) % N_DEV)

            x8_ref[...] = x_ref[...].astype(F8)

            bsem = pltpu.get_barrier_semaphore()
            for d in range(N_DEV):
                pl.semaphore_signal(
                    bsem, inc=1,
                    device_id=(d,), device_id_type=pl.DeviceIdType.MESH,
                )
            pl.semaphore_wait(bsem, N_DEV)

            recv_ref[me] = x8_ref[pl.ds(me * m_blk, m_blk), :]

            for off in range(1, N_DEV):
                d = (me + off) % N_DEV
                rdma = pltpu.make_async_remote_copy(
                    src_ref=x8_ref.at[pl.ds(d * m_blk, m_blk), :],
                    dst_ref=recv_ref.at[me],
                    send_sem=send_sems.at[d],
                    recv_sem=recv_sems.at[me],
                    device_id=(d,),
                    device_id_type=pl.DeviceIdType.MESH,
                )
                rdma.start()

        j = (me + c) % N_DEV
        r = c % R

        @pl.when(c != 0)
        def _wait_recv():
            recv = pltpu.make_async_remote_copy(
                src_ref=x8_ref.at[pl.ds(0, m_blk), :],
                dst_ref=recv_ref.at[j],
                send_sem=send_sems.at[j],
                recv_sem=recv_sems.at[j],
                device_id=(me,),
                device_id_type=pl.DeviceIdType.MESH,
            )
            recv.wait_recv()

        wait_slot(r, j)

        a = recv_ref[j]
        for h in range(H):
            w8 = wbuf_ref[r, :, pl.ds(h * n_h, n_h)].astype(F8)
            part = jnp.dot(a, w8, preferred_element_type=jnp.float32)

            @pl.when(c == 0)
            def _init():
                out_ref[:, pl.ds(h * n_h, n_h)] = part

            @pl.when(jnp.logical_and(c != 0, c != N_DEV - 1))
            def _accum():
                out_ref[:, pl.ds(h * n_h, n_h)] += part

            @pl.when(c == N_DEV - 1)
            def _fin():
                s = sx_ref[0] * sw_ref[0]
                out_ref[:, pl.ds(h * n_h, n_h)] = jnp.maximum(
                    (out_ref[:, pl.ds(h * n_h, n_h)] + part) * s, 0.0
                )

        @pl.when(c < N_DEV - R)
        def _refill():
            fill_slot(r, (me + c + R) % N_DEV)

        @pl.when(c == N_DEV - 1)
        def _drain():
            for off in range(1, N_DEV):
                d = (me + off) % N_DEV
                sd = pltpu.make_async_remote_copy(
                    src_ref=x8_ref.at[pl.ds(d * m_blk, m_blk), :],
                    dst_ref=recv_ref.at[me],
                    send_sem=send_sems.at[d],
                    recv_sem=recv_sems.at[me],
                    device_id=(d,),
                    device_id_type=pl.DeviceIdType.MESH,
                )
                sd.wait_send()

    return pl.pallas_call(
        body,
        grid=(N_DEV,),
        out_shape=jax.ShapeDtypeStruct((m_blk, n_total), jnp.float32),
        in_specs=[
            pl.BlockSpec((m_total, k_shard), lambda c: (0, 0)),
            pl.BlockSpec(memory_space=pl.ANY),
            pl.BlockSpec(memory_space=pltpu.SMEM),
            pl.BlockSpec(memory_space=pltpu.SMEM),
        ],
        out_specs=pl.BlockSpec((m_blk, n_total), lambda c: (0, 0)),
        scratch_shapes=[
            pltpu.VMEM((R, m_blk, n_total), jnp.float32),
            pltpu.VMEM((m_total, k_shard), F8),
            pltpu.VMEM((N_DEV, m_blk, k_shard), F8),
            pltpu.SemaphoreType.DMA((R, S)),
            pltpu.SemaphoreType.DMA((N_DEV,)),
            pltpu.SemaphoreType.DMA((N_DEV,)),
        ],
        compiler_params=pltpu.CompilerParams(
            dimension_semantics=("arbitrary",),
            collective_id=0,
            has_side_effects=True,
            vmem_limit_bytes=64 * 1024 * 1024,
        ),
    )(x, w_mat, scale_x, scale_w)


# device time: 57995 ns/iter; 1.2294x vs baseline; 1.0180x over previous
import jax
import jax.numpy as jnp
from jax import lax
from jax.experimental import pallas as pl
from jax.experimental.pallas import tpu as pltpu

N_DEV = 16
F8 = jnp.float8_e4m3fn
PAIR = 2
STEPS = N_DEV // PAIR
R = 2
S = 4
H = 2


def kernel(x, w_mat, scale_x, scale_w):
    m_total, k_shard = x.shape
    k_total, n_total = w_mat.shape
    m_blk = m_total // N_DEV
    rows = m_blk // S
    k_step = m_blk * PAIR
    n_h = n_total // H

    def body(x_ref, w_ref, sx_ref, sw_ref, out_ref,
             wbuf_ref, x8_ref, recv_ref, a_ref, wsems, send_sems, recv_sems):
        c = pl.program_id(0)
        me = lax.axis_index("i")

        def slot_dmas(slot, step):
            for p in range(PAIR):
                chunk = (me + PAIR * step + p) % N_DEV
                for i in range(S):
                    yield pltpu.make_async_copy(
                        w_ref.at[pl.ds(chunk * m_blk + i * rows, rows), :],
                        wbuf_ref.at[slot, pl.ds(p * m_blk + i * rows, rows), :],
                        wsems.at[slot, p * S + i],
                    )

        def fill_slot(slot, step):
            for d in slot_dmas(slot, step):
                d.start()

        def wait_slot(slot, step):
            for d in slot_dmas(slot, step):
                d.wait()

        @pl.when(c == 0)
        def _setup():
            for r in range(R):
                fill_slot(r, r)

            x8_ref[...] = x_ref[...].astype(F8)

            bsem = pltpu.get_barrier_semaphore()
            for d in range(N_DEV):
                pl.semaphore_signal(
                    bsem, inc=1,
                    device_id=(d,), device_id_type=pl.DeviceIdType.MESH,
                )
            pl.semaphore_wait(bsem, N_DEV)

            recv_ref[me] = x8_ref[pl.ds(me * m_blk, m_blk), :]

            for off in range(1, N_DEV):
                d = (me + off) % N_DEV
                rdma = pltpu.make_async_remote_copy(
                    src_ref=x8_ref.at[pl.ds(d * m_blk, m_blk), :],
                    dst_ref=recv_ref.at[me],
                    send_sem=send_sems.at[d],
                    recv_sem=recv_sems.at[me],
                    device_id=(d,),
                    device_id_type=pl.DeviceIdType.MESH,
                )
                rdma.start()

        r = c % R

        for p in range(PAIR):
            j = (me + PAIR * c + p) % N_DEV

            @pl.when(j != me)
            def _wait_recv():
                recv = pltpu.make_async_remote_copy(
                    src_ref=x8_ref.at[pl.ds(0, m_blk), :],
                    dst_ref=recv_ref.at[j],
                    send_sem=send_sems.at[j],
                    recv_sem=recv_sems.at[j],
                    device_id=(me,),
                    device_id_type=pl.DeviceIdType.MESH,
                )
                recv.wait_recv()

            a_ref[c % 2, :, pl.ds(p * m_blk, m_blk)] = recv_ref[j]

        wait_slot(r, c)

        a = a_ref[c % 2]
        for h in range(H):
            w8 = wbuf_ref[r, :, pl.ds(h * n_h, n_h)].astype(F8)
            part = jnp.dot(a, w8, preferred_element_type=jnp.float32)

            @pl.when(c == 0)
            def _init():
                out_ref[:, pl.ds(h * n_h, n_h)] = part

            @pl.when(jnp.logical_and(c != 0, c != STEPS - 1))
            def _accum():
                out_ref[:, pl.ds(h * n_h, n_h)] += part

            @pl.when(c == STEPS - 1)
            def _fin():
                s = sx_ref[0] * sw_ref[0]
                out_ref[:, pl.ds(h * n_h, n_h)] = jnp.maximum(
                    (out_ref[:, pl.ds(h * n_h, n_h)] + part) * s, 0.0
                )

        @pl.when(c < STEPS - R)
        def _refill():
            fill_slot(r, c + R)

        @pl.when(c == STEPS - 1)
        def _drain():
            for off in range(1, N_DEV):
                d = (me + off) % N_DEV
                sd = pltpu.make_async_remote_copy(
                    src_ref=x8_ref.at[pl.ds(d * m_blk, m_blk), :],
                    dst_ref=recv_ref.at[me],
                    send_sem=send_sems.at[d],
                    recv_sem=recv_sems.at[me],
                    device_id=(d,),
                    device_id_type=pl.DeviceIdType.MESH,
                )
                sd.wait_send()

    return pl.pallas_call(
        body,
        grid=(STEPS,),
        out_shape=jax.ShapeDtypeStruct((m_blk, n_total), jnp.float32),
        in_specs=[
            pl.BlockSpec((m_total, k_shard), lambda c: (0, 0)),
            pl.BlockSpec(memory_space=pl.ANY),
            pl.BlockSpec(memory_space=pltpu.SMEM),
            pl.BlockSpec(memory_space=pltpu.SMEM),
        ],
        out_specs=pl.BlockSpec((m_blk, n_total), lambda c: (0, 0)),
        scratch_shapes=[
            pltpu.VMEM((R, k_step, n_total), jnp.float32),
            pltpu.VMEM((m_total, k_shard), F8),
            pltpu.VMEM((N_DEV, m_blk, k_shard), F8),
            pltpu.VMEM((2, m_blk, k_step), F8),
            pltpu.SemaphoreType.DMA((R, PAIR * S)),
            pltpu.SemaphoreType.DMA((N_DEV,)),
            pltpu.SemaphoreType.DMA((N_DEV,)),
        ],
        compiler_params=pltpu.CompilerParams(
            dimension_semantics=("arbitrary",),
            collective_id=0,
            has_side_effects=True,
            vmem_limit_bytes=64 * 1024 * 1024,
        ),
    )(x, w_mat, scale_x, scale_w)
